# baseline (speedup 1.0000x reference)
import jax
import jax.numpy as jnp
from jax.experimental import pallas as pl
from jax.experimental.pallas import tpu as pltpu

B, N, D = 8, 2048, 32


def _gcn_fused_kernel(
    text_hbm, adj_ref, w_ref, b_ref, out_ref, text_vmem, hidden_ref, sem
):
    bi = pl.program_id(0)

    @pl.when(bi == 0)
    def _():
        copy = pltpu.make_async_copy(text_hbm, text_vmem, sem)
        copy.start()
        copy.wait()
        hidden_ref[:, :D] = jnp.dot(
            text_vmem[...], w_ref[...], preferred_element_type=jnp.float32
        )
        hidden_ref[:, D:] = jnp.ones((B * N, D), jnp.float32)

    a = adj_ref[0]
    h = hidden_ref[pl.ds(bi * N, N), :]
    acc = jnp.dot(a, h, preferred_element_type=jnp.float32)
    out_ref[0] = acc[:, :D] / (acc[:, D:] + 1.0) + b_ref[...]


def kernel(text, adj, W, b):
    b2d = b.reshape(1, D)
    text2d = text.reshape(B * N, D)
    return pl.pallas_call(
        _gcn_fused_kernel,
        grid=(B,),
        in_specs=[
            pl.BlockSpec(memory_space=pltpu.MemorySpace.HBM),
            pl.BlockSpec((1, N, N), lambda bi: (bi, 0, 0)),
            pl.BlockSpec((D, D), lambda bi: (0, 0)),
            pl.BlockSpec((1, D), lambda bi: (0, 0)),
        ],
        out_specs=pl.BlockSpec((1, N, D), lambda bi: (bi, 0, 0)),
        out_shape=jax.ShapeDtypeStruct((B, N, D), jnp.float32),
        scratch_shapes=[
            pltpu.VMEM((B * N, D), jnp.float32),
            pltpu.VMEM((B * N, 2 * D), jnp.float32),
            pltpu.SemaphoreType.DMA,
        ],
        compiler_params=pltpu.CompilerParams(
            dimension_semantics=("arbitrary",),
        ),
    )(text2d, adj, W, b2d)


# speedup vs baseline: 1.0538x; 1.0538x over previous
import jax
import jax.numpy as jnp
from jax.experimental import pallas as pl
from jax.experimental.pallas import tpu as pltpu

B, N, D = 8, 2048, 32


def _gcn_fused_kernel(text_ref, adj_ref, w_ref, b_ref, out_ref, hidden_ref):
    hidden_ref[:, :D] = jnp.dot(
        text_ref[0], w_ref[...], preferred_element_type=jnp.float32
    )
    hidden_ref[:, D:] = jnp.ones((N, D), jnp.float32)

    a = adj_ref[0]
    acc = jnp.dot(a, hidden_ref[...], preferred_element_type=jnp.float32)
    out_ref[0] = acc[:, :D] / (acc[:, D:] + 1.0) + b_ref[...]


def kernel(text, adj, W, b):
    b2d = b.reshape(1, D)
    return pl.pallas_call(
        _gcn_fused_kernel,
        grid=(B,),
        in_specs=[
            pl.BlockSpec((1, N, D), lambda bi: (bi, 0, 0)),
            pl.BlockSpec((1, N, N), lambda bi: (bi, 0, 0)),
            pl.BlockSpec((D, D), lambda bi: (0, 0)),
            pl.BlockSpec((1, D), lambda bi: (0, 0)),
        ],
        out_specs=pl.BlockSpec((1, N, D), lambda bi: (bi, 0, 0)),
        out_shape=jax.ShapeDtypeStruct((B, N, D), jnp.float32),
        scratch_shapes=[pltpu.VMEM((N, 2 * D), jnp.float32)],
        compiler_params=pltpu.CompilerParams(
            dimension_semantics=("arbitrary",),
        ),
    )(text, adj, W, b2d)
